# Initial kernel scaffold; baseline (speedup 1.0000x reference)
import jax
import jax.numpy as jnp
from jax.experimental import pallas as pl


def kernel(token_ids, weight):
    raise NotImplementedError("write your pallas kernel here")


# speedup vs baseline: 1.7981x; 1.7981x over previous
import functools

import jax
import jax.numpy as jnp
from jax import lax
from jax.experimental import pallas as pl
from jax.experimental.pallas import tpu as pltpu
from jax.experimental.pallas import tpu_sc as plsc

NUM_EMB = 1_000_000
DIM = 64

NC = 2
NS = 16
NW = NC * NS

B_TOK, T_TOK = 16384, 50
N_FLAT = B_TOK * T_TOK
PER_W = N_FLAT // NW
SUB = 128
K = 4
CHUNK = SUB * K
N_ITER = PER_W // CHUNK
IDX_ROWS_PER_W = PER_W // SUB

_mesh = plsc.VectorSubcoreMesh(
    core_axis_name="c", subcore_axis_name="s", num_cores=NC, num_subcores=NS
)


@functools.partial(
    pl.kernel,
    out_type=jax.ShapeDtypeStruct((N_FLAT, DIM), jnp.float32),
    mesh=_mesh,
    scratch_types=[
        pltpu.VMEM((K, SUB), jnp.int32),
        pltpu.VMEM((CHUNK, DIM), jnp.float32),
        pltpu.SemaphoreType.DMA,
    ],
    compiler_params=pltpu.CompilerParams(use_tc_tiling_on_sc=False),
)
def _emb_gather(table_hbm, idx_hbm, out_hbm, idx_v, rows_v, sem):
    wid = lax.axis_index("s") * NC + lax.axis_index("c")
    out_base = wid * PER_W
    idx_row_base = wid * IDX_ROWS_PER_W

    @pl.loop(0, N_ITER)
    def _body(i):
        pltpu.sync_copy(idx_hbm.at[pl.ds(idx_row_base + i * K, K)], idx_v)
        copies = []
        for k in range(K):
            copies.append(
                pltpu.async_copy(
                    table_hbm.at[idx_v.at[k]],
                    rows_v.at[pl.ds(k * SUB, SUB)],
                    sem,
                )
            )
        for c in copies:
            c.wait()
        pltpu.sync_copy(rows_v, out_hbm.at[pl.ds(out_base + i * CHUNK, CHUNK)])


def kernel(token_ids, weight):
    idx2d = token_ids.reshape(N_FLAT // SUB, SUB).astype(jnp.int32)
    out = _emb_gather(weight, idx2d)
    return out.reshape(B_TOK, T_TOK, DIM)


# speedup vs baseline: 1.8732x; 1.0418x over previous
import functools

import jax
import jax.numpy as jnp
from jax import lax
from jax.experimental import pallas as pl
from jax.experimental.pallas import tpu as pltpu
from jax.experimental.pallas import tpu_sc as plsc

NUM_EMB = 1_000_000
DIM = 64

NC = 2
NS = 16
NW = NC * NS

B_TOK, T_TOK = 16384, 50
N_FLAT = B_TOK * T_TOK
PER_W = N_FLAT // NW
SUB = 128
K = 4
CHUNK = SUB * K
N_ITER = PER_W // CHUNK
IDX_ROWS_PER_W = PER_W // SUB
NBUF = 3

_mesh = plsc.VectorSubcoreMesh(
    core_axis_name="c", subcore_axis_name="s", num_cores=NC, num_subcores=NS
)


@functools.partial(
    pl.kernel,
    out_type=jax.ShapeDtypeStruct((N_FLAT, DIM), jnp.float32),
    mesh=_mesh,
    scratch_types=[
        pltpu.VMEM((IDX_ROWS_PER_W, SUB), jnp.int32),
        pltpu.VMEM((NBUF, CHUNK, DIM), jnp.float32),
        pltpu.SemaphoreType.DMA((NBUF,)),
        pltpu.SemaphoreType.DMA((NBUF,)),
    ],
    compiler_params=pltpu.CompilerParams(use_tc_tiling_on_sc=False),
)
def _emb_gather(table_hbm, idx_hbm, out_hbm, idx_v, rows_v, sem_g, sem_o):
    wid = lax.axis_index("s") * NC + lax.axis_index("c")
    out_base = wid * PER_W

    pltpu.sync_copy(idx_hbm.at[pl.ds(wid * IDX_ROWS_PER_W, IDX_ROWS_PER_W)], idx_v)

    def fire_g(i, slot):
        for k in range(K):
            pltpu.async_copy(
                table_hbm.at[idx_v.at[i * K + k]],
                rows_v.at[slot].at[pl.ds(k * SUB, SUB)],
                sem_g.at[slot],
            )

    def drain_g(slot):
        pltpu.make_async_copy(
            out_hbm.at[pl.ds(0, CHUNK)], rows_v.at[slot], sem_g.at[slot]
        ).wait()

    def fire_w(i, slot):
        pltpu.async_copy(
            rows_v.at[slot], out_hbm.at[pl.ds(out_base + i * CHUNK, CHUNK)],
            sem_o.at[slot],
        )

    def wait_w(slot):
        pltpu.make_async_copy(
            rows_v.at[slot], out_hbm.at[pl.ds(0, CHUNK)], sem_o.at[slot]
        ).wait()

    fire_g(0, 0)
    fire_g(1, 1)
    drain_g(0)
    fire_w(0, 0)
    fire_g(2, 2)

    @pl.loop(1, 46, step=NBUF)
    def _grp(i0):
        for d in range(NBUF):
            i = i0 + d
            slot = (1 + d) % NBUF
            drain_g(slot)
            fire_w(i, slot)
            wait_w((slot + 2) % NBUF)
            fire_g(i + 2, (slot + 2) % NBUF)

    for i, fire in ((46, True), (47, True), (48, False), (49, False)):
        slot = i % NBUF
        drain_g(slot)
        fire_w(i, slot)
        if fire:
            wait_w((slot + 2) % NBUF)
            fire_g(i + 2, (slot + 2) % NBUF)
    for slot in ((47 % NBUF), (48 % NBUF), (49 % NBUF)):
        wait_w(slot)


def kernel(token_ids, weight):
    idx2d = token_ids.reshape(N_FLAT // SUB, SUB).astype(jnp.int32)
    out = _emb_gather(weight, idx2d)
    return out.reshape(B_TOK, T_TOK, DIM)
